# baseline (speedup 1.0000x reference)
import functools

import jax
import jax.numpy as jnp
from jax import lax
from jax.experimental import pallas as pl
from jax.experimental.pallas import tpu as pltpu
from jax.experimental.pallas import tpu_sc as plsc

N = 10000
H = 128
D = 192
T = 4
E = 80000
ET = T * E
SW = 144
NW = 32
EW = ET // NW
FMIN = float(jnp.finfo(jnp.float32).min)
EPS = 1e-07

GCH = 80
GNC = (2 * EW) // GCH
SCH = 80
SNC = EW // SCH
MCH = 2000
MNC = ET // MCH
BAT = 5

_PREC = lax.Precision.HIGHEST
_SC_PARAMS = pltpu.CompilerParams(use_tc_tiling_on_sc=False)
_SC_PARAMS_NL = pltpu.CompilerParams(
    use_tc_tiling_on_sc=False, needs_layout_passes=False
)


def _mesh():
    return plsc.VectorSubcoreMesh(core_axis_name="c", subcore_axis_name="s")


def _k1_body(ns_ref, w0_ref, ab_ref):
    x = ns_ref[...]
    w = w0_ref[0]
    ab_ref[0, 0] = jnp.dot(x, w[:H], precision=_PREC)
    ab_ref[1, 0] = jnp.dot(x, w[H:], precision=_PREC)


def _k1(ns, w0):
    nb = 2000
    return pl.pallas_call(
        _k1_body,
        grid=(T, N // nb),
        in_specs=[
            pl.BlockSpec((nb, H), lambda t, i: (i, 0)),
            pl.BlockSpec((1, 2 * H, D), lambda t, i: (t, 0, 0)),
        ],
        out_specs=pl.BlockSpec((2, 1, nb, D), lambda t, i: (0, t, i, 0)),
        out_shape=jax.ShapeDtypeStruct((2, T, N, D), jnp.float32),
    )(ns, w0)


def _k2_body(tab_hbm, idx_hbm, out_hbm, idx_v, b0, b1, o0, o1, s0, s1):
    wid = lax.axis_index("s") * 2 + lax.axis_index("c")
    pltpu.sync_copy(idx_hbm.at[wid], idx_v)
    ebase = wid * EW
    bufs = (b0, b1)
    obufs = (o0, o1)
    sems = (s0, s1)
    ge = GCH // 2

    for b in range(2):
        pltpu.async_copy(tab_hbm.at[idx_v.at[b]], bufs[b], sems[b])

    def outer(o, carry):
        for b in range(2):
            j = o * 2 + b
            pltpu.make_async_copy(
                tab_hbm.at[idx_v.at[j]], bufs[b], sems[b]
            ).wait()

            def add_row(r, carry2):
                for c in range(D // 16):
                    sl = pl.ds(c * 16, 16)
                    obufs[b][r, sl] = jnp.maximum(
                        bufs[b][2 * r, sl] + bufs[b][2 * r + 1, sl], 0.0
                    )
                return carry2

            lax.fori_loop(0, ge, add_row, 0, unroll=2)

            @pl.when(o < GNC // 2 - 1)
            def _():
                pltpu.async_copy(
                    tab_hbm.at[idx_v.at[j + 2]], bufs[b], sems[b]
                )

            pltpu.sync_copy(
                obufs[b], out_hbm.at[pl.ds(ebase + j * ge, ge)]
            )
        return carry

    lax.fori_loop(0, GNC // 2, outer, 0)


def _k2(tab, idx2):
    f = pl.kernel(
        _k2_body,
        compiler_params=_SC_PARAMS,
        out_type=jax.ShapeDtypeStruct((ET, D), jnp.float32),
        mesh=_mesh(),
        scratch_types=[
            pltpu.VMEM((GNC, GCH), jnp.int32),
            pltpu.VMEM((GCH, D), jnp.float32),
            pltpu.VMEM((GCH, D), jnp.float32),
            pltpu.VMEM((GCH // 2, D), jnp.float32),
            pltpu.VMEM((GCH // 2, D), jnp.float32),
            pltpu.SemaphoreType.DMA,
            pltpu.SemaphoreType.DMA,
        ],
    )
    return f(tab, idx2)


def _k3_body(p_ref, w1_ref, s_ref, x_ref):
    x = p_ref[...]
    h = jnp.maximum(jnp.dot(x, w1_ref[0]), 0.0)
    eb = h.shape[0]
    tail = jnp.broadcast_to(
        (jnp.arange(16) == 0).astype(jnp.float32)[None, :], (eb, 16)
    )
    s_ref[...] = jnp.concatenate([h[:, :2 * 64], tail], axis=1)
    x_ref[...] = h[:, 2 * 64:].T


def _k3(p2, w1):
    eb = 3200
    ebt = E // eb
    return pl.pallas_call(
        _k3_body,
        grid=(T, ebt),
        in_specs=[
            pl.BlockSpec((eb, D), lambda t, i: (t * ebt + i, 0)),
            pl.BlockSpec((1, D, D), lambda t, i: (t, 0, 0)),
        ],
        out_specs=[
            pl.BlockSpec((eb, SW), lambda t, i: (t * ebt + i, 0)),
            pl.BlockSpec((64, eb), lambda t, i: (0, t * ebt + i)),
        ],
        out_shape=[
            jax.ShapeDtypeStruct((ET, SW), jnp.float32),
            jax.ShapeDtypeStruct((64, ET), jnp.float32),
        ],
    )(p2, w1)


def _k4_body(s_hbm, tgt_hbm, out_hbm, acc, idx_v, sb0, sb1, sem0, sem1):
    cid = lax.axis_index("c")
    sid = lax.axis_index("s")
    wid = sid * 2 + cid
    rows_per_tile = N // 16

    def zrow(r, carry):
        for c in range(SW // 16):
            sb0[r, pl.ds(c * 16, 16)] = jnp.zeros((16,), jnp.float32)
        return carry

    lax.fori_loop(0, SCH, zrow, 0, unroll=4)
    for z in range(rows_per_tile // SCH):
        pltpu.sync_copy(sb0, acc.at[pl.ds(sid * rows_per_tile + z * SCH, SCH)])
    rem = rows_per_tile % SCH
    pltpu.sync_copy(
        sb0.at[pl.ds(0, rem)],
        acc.at[pl.ds(sid * rows_per_tile + rows_per_tile - rem, rem)],
    )
    plsc.subcore_barrier()

    pltpu.sync_copy(tgt_hbm.at[wid], idx_v)

    sbufs = (sb0, sb1)
    sems = (sem0, sem1)
    for b in range(2):
        pltpu.async_copy(
            s_hbm.at[pl.ds(wid * EW + b * SCH, SCH)], sbufs[b], sems[b]
        )

    def body(o, carry):
        for b in range(2):
            j = o * 2 + b
            pltpu.make_async_copy(
                s_hbm.at[pl.ds(wid * EW + j * SCH, SCH)], sbufs[b], sems[b]
            ).wait()
            pltpu.sync_copy(sbufs[b], acc.at[idx_v.at[j]], add=True)

            @pl.when(o < SNC // 2 - 1)
            def _():
                pltpu.async_copy(
                    s_hbm.at[pl.ds(wid * EW + (j + 2) * SCH, SCH)],
                    sbufs[b], sems[b],
                )
        return carry

    lax.fori_loop(0, SNC // 2, body, 0)
    j_last = SNC - 1
    pltpu.async_copy(
        s_hbm.at[pl.ds(wid * EW + j_last * SCH, SCH)], sb0, sem0
    ).wait()
    pltpu.sync_copy(sb0, acc.at[idx_v.at[j_last]], add=True)

    plsc.subcore_barrier()
    pltpu.sync_copy(
        acc.at[pl.ds(sid * rows_per_tile, rows_per_tile)],
        out_hbm.at[cid, pl.ds(sid * rows_per_tile, rows_per_tile)],
    )


def _k4(s, tgt3):
    f = pl.kernel(
        _k4_body,
        compiler_params=_SC_PARAMS,
        out_type=jax.ShapeDtypeStruct((2, N, SW), jnp.float32),
        mesh=_mesh(),
        scratch_types=[
            pltpu.VMEM_SHARED((N, SW), jnp.float32),
            pltpu.VMEM((SNC, SCH), jnp.int32),
            pltpu.VMEM((SCH, SW), jnp.float32),
            pltpu.VMEM((SCH, SW), jnp.float32),
            pltpu.SemaphoreType.DMA,
            pltpu.SemaphoreType.DMA,
        ],
    )
    return f(s, tgt3)


def _k57_body(x_hbm, tgtf_hbm, s_hbm, tgt3_hbm, mean_hbm, xout_hbm,
              stdout_hbm, acc7, idx_v, mb0, mb1, ub0, ub1, acc5, tb0, tb1,
              xb0, xb1, sem0, sem1, sem2, sem3):
    cid = lax.axis_index("c")
    sid = lax.axis_index("s")
    wid = sid * 2 + cid
    rows_per_tile = N // 16

    def zrow(r, carry):
        for c in range(4):
            mb0[r, pl.ds(c * 16, 16)] = jnp.zeros((16,), jnp.float32)
        return carry

    lax.fori_loop(0, SCH, zrow, 0, unroll=4)
    for z in range(rows_per_tile // SCH):
        pltpu.sync_copy(
            mb0, acc7.at[pl.ds(sid * rows_per_tile + z * SCH, SCH)]
        )
    rem = rows_per_tile % SCH
    pltpu.sync_copy(
        mb0.at[pl.ds(0, rem)],
        acc7.at[pl.ds(sid * rows_per_tile + rows_per_tile - rem, rem)],
    )
    plsc.subcore_barrier()

    def irow(i, carry):
        for c in range(2):
            acc5[c, pl.ds(i * 16, 16)] = jnp.full((16,), FMIN, jnp.float32)
        return carry

    lax.fori_loop(0, N // 16, irow, 0, unroll=8)

    tbufs = (tb0, tb1)
    xbufs = (xb0, xb1)
    tsems = (sem0, sem1)
    xsems = (sem2, sem3)

    for b in range(2):
        pltpu.async_copy(tgtf_hbm.at[pl.ds(b * MCH, MCH)], tbufs[b], tsems[b])
        pltpu.async_copy(
            x_hbm.at[pl.ds(2 * wid, 2), pl.ds(b * MCH, MCH)], xbufs[b],
            xsems[b],
        )

    cvecs = [jnp.full((16,), c, jnp.int32) for c in range(2)]

    def chunk(o, carry):
        for b in range(2):
            q = o * 2 + b
            pltpu.make_async_copy(
                tgtf_hbm.at[pl.ds(q * MCH, MCH)], tbufs[b], tsems[b]
            ).wait()
            pltpu.make_async_copy(
                x_hbm.at[pl.ds(2 * wid, 2), pl.ds(q * MCH, MCH)], xbufs[b],
                xsems[b],
            ).wait()

            def inner(k, carry2):
                pairs = []
                for i in range(BAT):
                    sl = pl.ds((k * BAT + i) * 16, 16)
                    idx = tbufs[b][sl]
                    for c in range(2):
                        pairs.append((cvecs[c], idx, xbufs[b][c, sl]))

                gs = [plsc.load_gather(acc5, [cv, ix]) for cv, ix, _ in pairs]
                for (cv, ix, v), g in zip(pairs, gs):
                    plsc.store_scatter(
                        acc5, [cv, ix], jnp.maximum(g, v), mask=v > g
                    )

                def cond(flag):
                    return flag

                def recheck(flag):
                    g2s = [
                        plsc.load_gather(acc5, [cv, ix])
                        for cv, ix, _ in pairs
                    ]
                    imps = [v > g2 for (_, _, v), g2 in zip(pairs, g2s)]
                    for (cv, ix, v), g2, imp in zip(pairs, g2s, imps):
                        plsc.store_scatter(
                            acc5, [cv, ix], jnp.maximum(g2, v), mask=imp
                        )
                    m = imps[0]
                    for im in imps[1:]:
                        m = jnp.logical_or(m, im)
                    return jnp.any(m)

                lax.while_loop(cond, recheck, True)
                return carry2

            lax.fori_loop(0, (MCH // 16) // BAT, inner, 0)

            @pl.when(o < MNC // 2 - 1)
            def _():
                pltpu.async_copy(
                    tgtf_hbm.at[pl.ds((q + 2) * MCH, MCH)], tbufs[b],
                    tsems[b],
                )
                pltpu.async_copy(
                    x_hbm.at[pl.ds(2 * wid, 2), pl.ds((q + 2) * MCH, MCH)],
                    xbufs[b], xsems[b],
                )
        return carry

    lax.fori_loop(0, MNC // 2, chunk, 0)
    pltpu.sync_copy(acc5, xout_hbm.at[pl.ds(2 * wid, 2)])

    pltpu.sync_copy(tgt3_hbm.at[wid], idx_v)

    mbufs = (mb0, mb1)
    ubufs = (ub0, ub1)
    msems = (sem0, sem1)
    usems = (sem2, sem3)

    for b in range(2):
        pltpu.async_copy(
            s_hbm.at[pl.ds(wid * EW + b * SCH, SCH), pl.ds(64, 64)],
            mbufs[b], msems[b],
        )
        pltpu.async_copy(mean_hbm.at[idx_v.at[b]], ubufs[b], usems[b])

    def sbody(o, carry):
        for b in range(2):
            j = o * 2 + b
            pltpu.make_async_copy(
                s_hbm.at[pl.ds(wid * EW + j * SCH, SCH), pl.ds(64, 64)],
                mbufs[b], msems[b],
            ).wait()
            pltpu.make_async_copy(
                mean_hbm.at[idx_v.at[j]], ubufs[b], usems[b]
            ).wait()

            def crow(r, carry2):
                for c in range(4):
                    sl = pl.ds(c * 16, 16)
                    m = mbufs[b][r, sl]
                    u = ubufs[b][r, sl]
                    mbufs[b][r, sl] = jnp.maximum(m * m - u * u, 0.0) + EPS
                return carry2

            lax.fori_loop(0, SCH, crow, 0, unroll=4)
            pltpu.sync_copy(mbufs[b], acc7.at[idx_v.at[j]], add=True)

            @pl.when(o < SNC // 2 - 1)
            def _():
                pltpu.async_copy(
                    s_hbm.at[pl.ds(wid * EW + (j + 2) * SCH, SCH),
                             pl.ds(64, 64)],
                    mbufs[b], msems[b],
                )
                pltpu.async_copy(
                    mean_hbm.at[idx_v.at[j + 2]], ubufs[b], usems[b]
                )
        return carry

    lax.fori_loop(0, SNC // 2, sbody, 0)
    j_last = SNC - 1
    pltpu.async_copy(
        s_hbm.at[pl.ds(wid * EW + j_last * SCH, SCH), pl.ds(64, 64)],
        mb0, sem0,
    ).wait()
    pltpu.async_copy(mean_hbm.at[idx_v.at[j_last]], ub0, sem2).wait()

    def crow_last(r, carry2):
        for c in range(4):
            sl = pl.ds(c * 16, 16)
            m = mb0[r, sl]
            u = ub0[r, sl]
            mb0[r, sl] = jnp.maximum(m * m - u * u, 0.0) + EPS
        return carry2

    lax.fori_loop(0, SCH, crow_last, 0, unroll=4)
    pltpu.sync_copy(mb0, acc7.at[idx_v.at[j_last]], add=True)

    plsc.subcore_barrier()
    pltpu.sync_copy(
        acc7.at[pl.ds(sid * rows_per_tile, rows_per_tile)],
        stdout_hbm.at[cid, pl.ds(sid * rows_per_tile, rows_per_tile)],
    )


def _k57(x, tgtf, s, tgt3, mean):
    f = pl.kernel(
        _k57_body,
        compiler_params=_SC_PARAMS_NL,
        out_type=(
            jax.ShapeDtypeStruct((64, N), jnp.float32),
            jax.ShapeDtypeStruct((2, N, 64), jnp.float32),
        ),
        mesh=_mesh(),
        scratch_types=[
            pltpu.VMEM_SHARED((N, 64), jnp.float32),
            pltpu.VMEM((SNC, SCH), jnp.int32),
            pltpu.VMEM((SCH, 64), jnp.float32),
            pltpu.VMEM((SCH, 64), jnp.float32),
            pltpu.VMEM((SCH, 64), jnp.float32),
            pltpu.VMEM((SCH, 64), jnp.float32),
            pltpu.VMEM((2, N), jnp.float32),
            pltpu.VMEM((MCH,), jnp.int32),
            pltpu.VMEM((MCH,), jnp.int32),
            pltpu.VMEM((2, MCH), jnp.float32),
            pltpu.VMEM((2, MCH), jnp.float32),
            pltpu.SemaphoreType.DMA,
            pltpu.SemaphoreType.DMA,
            pltpu.SemaphoreType.DMA,
            pltpu.SemaphoreType.DMA,
        ],
    )
    return f(x, tgtf, s, tgt3, mean)


def _k6_body(part_ref, sum_ref, mean_ref):
    s = part_ref[0] + part_ref[1]
    sum_ref[...] = s[:, :64]
    mean_ref[...] = s[:, 64:128] / s[:, 128:129]


def _k6(part):
    nb = 1000
    return pl.pallas_call(
        _k6_body,
        grid=(N // nb,),
        in_specs=[pl.BlockSpec((2, nb, SW), lambda i: (0, i, 0))],
        out_specs=[
            pl.BlockSpec((nb, 64), lambda i: (i, 0)),
            pl.BlockSpec((nb, 64), lambda i: (i, 0)),
        ],
        out_shape=[
            jax.ShapeDtypeStruct((N, 64), jnp.float32),
            jax.ShapeDtypeStruct((N, 64), jnp.float32),
        ],
    )(part)


def _k8_body(sum_ref, mean_ref, sp_ref, x_ref, out_ref):
    std = jnp.sqrt(sp_ref[0] + sp_ref[1])
    out_ref[...] = jnp.concatenate(
        [sum_ref[...], mean_ref[...], std, x_ref[...].T], axis=1
    )


def _k8(sum_agg, mean, stdpart, xacc):
    return pl.pallas_call(
        _k8_body,
        out_shape=jax.ShapeDtypeStruct((N, 4 * 64), jnp.float32),
    )(sum_agg, mean, stdpart, xacc)


def kernel(node_states, adj0, adj1, adj2, adj3, W0, W1):
    adjs = [adj0, adj1, adj2, adj3]
    srcs = [a[:, 0].astype(jnp.int32) for a in adjs]
    tgts = [a[:, 1].astype(jnp.int32) for a in adjs]

    isrc = jnp.concatenate([s + t * N for t, s in enumerate(srcs)])
    itgt = jnp.concatenate([g + (T + t) * N for t, g in enumerate(tgts)])
    idx2 = jnp.stack([isrc, itgt], axis=1).reshape(NW, GNC, GCH)

    tgt_all = jnp.concatenate(tgts)
    tgt3 = tgt_all.reshape(NW, SNC, SCH)

    ab = _k1(node_states, W0)
    tab = ab.reshape(2 * T * N, D)
    p2 = _k2(tab, idx2)
    s, x = _k3(p2, W1)
    part = _k4(s, tgt3)
    sum_agg, mean = _k6(part)
    xacc, stdpart = _k57(x, tgt_all, s, tgt3, mean)
    return _k8(sum_agg, mean, stdpart, xacc)


# speedup vs baseline: 1.0796x; 1.0796x over previous
import functools

import jax
import jax.numpy as jnp
from jax import lax
from jax.experimental import pallas as pl
from jax.experimental.pallas import tpu as pltpu
from jax.experimental.pallas import tpu_sc as plsc

N = 10000
H = 128
D = 192
T = 4
E = 80000
ET = T * E
SW = 144
NW = 32
EW = ET // NW
FMIN = float(jnp.finfo(jnp.float32).min)
EPS = 1e-07

GCH = 125
GNC = (2 * EW) // GCH
GNB = 4
SCH = 80
SNC = EW // SCH
MCH = 2000
MNC = ET // MCH
BAT = 5

_PREC = lax.Precision.HIGHEST
_SC_PARAMS = pltpu.CompilerParams(use_tc_tiling_on_sc=False)
_SC_PARAMS_NL = pltpu.CompilerParams(
    use_tc_tiling_on_sc=False, needs_layout_passes=False
)


def _mesh():
    return plsc.VectorSubcoreMesh(core_axis_name="c", subcore_axis_name="s")


def _k1_body(ns_ref, w0_ref, ab_ref):
    x = ns_ref[...]
    w = w0_ref[0]
    ab_ref[0, 0] = jnp.dot(x, w[:H], precision=_PREC)
    ab_ref[1, 0] = jnp.dot(x, w[H:], precision=_PREC)


def _k1(ns, w0):
    nb = 2000
    return pl.pallas_call(
        _k1_body,
        grid=(T, N // nb),
        in_specs=[
            pl.BlockSpec((nb, H), lambda t, i: (i, 0)),
            pl.BlockSpec((1, 2 * H, D), lambda t, i: (t, 0, 0)),
        ],
        out_specs=pl.BlockSpec((2, 1, nb, D), lambda t, i: (0, t, i, 0)),
        out_shape=jax.ShapeDtypeStruct((2, T, N, D), jnp.float32),
    )(ns, w0)


def _k2_body(tab_hbm, idx_hbm, out_hbm, idx_v, b0, b1, b2, b3, s0, s1, s2, s3):
    wid = lax.axis_index("s") * 2 + lax.axis_index("c")
    pltpu.sync_copy(idx_hbm.at[wid], idx_v)
    base = wid * 2 * EW
    bufs = (b0, b1, b2, b3)
    sems = (s0, s1, s2, s3)

    for b in range(GNB):
        pltpu.async_copy(tab_hbm.at[idx_v.at[b]], bufs[b], sems[b])

    n_outer = GNC // GNB

    def outer(o, carry):
        for b in range(GNB):
            j = o * GNB + b
            pltpu.make_async_copy(
                tab_hbm.at[idx_v.at[j]], bufs[b], sems[b]
            ).wait()
            pltpu.sync_copy(bufs[b], out_hbm.at[pl.ds(base + j * GCH, GCH)])

            @pl.when(o < n_outer - 1)
            def _():
                pltpu.async_copy(
                    tab_hbm.at[idx_v.at[j + GNB]], bufs[b], sems[b]
                )
        return carry

    lax.fori_loop(0, n_outer, outer, 0)


def _k2(tab, idx2):
    f = pl.kernel(
        _k2_body,
        compiler_params=_SC_PARAMS,
        out_type=jax.ShapeDtypeStruct((2 * ET, D), jnp.float32),
        mesh=_mesh(),
        scratch_types=[
            pltpu.VMEM((GNC, GCH), jnp.int32),
            pltpu.VMEM((GCH, D), jnp.float32),
            pltpu.VMEM((GCH, D), jnp.float32),
            pltpu.VMEM((GCH, D), jnp.float32),
            pltpu.VMEM((GCH, D), jnp.float32),
            pltpu.SemaphoreType.DMA,
            pltpu.SemaphoreType.DMA,
            pltpu.SemaphoreType.DMA,
            pltpu.SemaphoreType.DMA,
        ],
    )
    return f(tab, idx2)


def _k3_body(p_ref, w1_ref, s_ref, x_ref):
    p = p_ref[...]
    x = jnp.maximum(p[:, :D] + p[:, D:], 0.0)
    h = jnp.maximum(jnp.dot(x, w1_ref[0]), 0.0)
    eb = h.shape[0]
    tail = jnp.broadcast_to(
        (jnp.arange(16) == 0).astype(jnp.float32)[None, :], (eb, 16)
    )
    s_ref[...] = jnp.concatenate([h[:, :2 * 64], tail], axis=1)
    x_ref[...] = h[:, 2 * 64:].T


def _k3(p2, w1):
    eb = 3200
    ebt = E // eb
    return pl.pallas_call(
        _k3_body,
        grid=(T, ebt),
        in_specs=[
            pl.BlockSpec((eb, 2 * D), lambda t, i: (t * ebt + i, 0)),
            pl.BlockSpec((1, D, D), lambda t, i: (t, 0, 0)),
        ],
        out_specs=[
            pl.BlockSpec((eb, SW), lambda t, i: (t * ebt + i, 0)),
            pl.BlockSpec((64, eb), lambda t, i: (0, t * ebt + i)),
        ],
        out_shape=[
            jax.ShapeDtypeStruct((ET, SW), jnp.float32),
            jax.ShapeDtypeStruct((64, ET), jnp.float32),
        ],
    )(p2, w1)


def _k4_body(s_hbm, tgt_hbm, out_hbm, acc, idx_v, sb0, sb1, sem0, sem1):
    cid = lax.axis_index("c")
    sid = lax.axis_index("s")
    wid = sid * 2 + cid
    rows_per_tile = N // 16

    def zrow(r, carry):
        for c in range(SW // 16):
            sb0[r, pl.ds(c * 16, 16)] = jnp.zeros((16,), jnp.float32)
        return carry

    lax.fori_loop(0, SCH, zrow, 0, unroll=4)
    for z in range(rows_per_tile // SCH):
        pltpu.sync_copy(sb0, acc.at[pl.ds(sid * rows_per_tile + z * SCH, SCH)])
    rem = rows_per_tile % SCH
    pltpu.sync_copy(
        sb0.at[pl.ds(0, rem)],
        acc.at[pl.ds(sid * rows_per_tile + rows_per_tile - rem, rem)],
    )
    plsc.subcore_barrier()

    pltpu.sync_copy(tgt_hbm.at[wid], idx_v)

    sbufs = (sb0, sb1)
    sems = (sem0, sem1)
    for b in range(2):
        pltpu.async_copy(
            s_hbm.at[pl.ds(wid * EW + b * SCH, SCH)], sbufs[b], sems[b]
        )

    def body(o, carry):
        for b in range(2):
            j = o * 2 + b
            pltpu.make_async_copy(
                s_hbm.at[pl.ds(wid * EW + j * SCH, SCH)], sbufs[b], sems[b]
            ).wait()
            pltpu.sync_copy(sbufs[b], acc.at[idx_v.at[j]], add=True)

            @pl.when(o < SNC // 2 - 1)
            def _():
                pltpu.async_copy(
                    s_hbm.at[pl.ds(wid * EW + (j + 2) * SCH, SCH)],
                    sbufs[b], sems[b],
                )
        return carry

    lax.fori_loop(0, SNC // 2, body, 0)
    j_last = SNC - 1
    pltpu.async_copy(
        s_hbm.at[pl.ds(wid * EW + j_last * SCH, SCH)], sb0, sem0
    ).wait()
    pltpu.sync_copy(sb0, acc.at[idx_v.at[j_last]], add=True)

    plsc.subcore_barrier()
    pltpu.sync_copy(
        acc.at[pl.ds(sid * rows_per_tile, rows_per_tile)],
        out_hbm.at[cid, pl.ds(sid * rows_per_tile, rows_per_tile)],
    )


def _k4(s, tgt3):
    f = pl.kernel(
        _k4_body,
        compiler_params=_SC_PARAMS,
        out_type=jax.ShapeDtypeStruct((2, N, SW), jnp.float32),
        mesh=_mesh(),
        scratch_types=[
            pltpu.VMEM_SHARED((N, SW), jnp.float32),
            pltpu.VMEM((SNC, SCH), jnp.int32),
            pltpu.VMEM((SCH, SW), jnp.float32),
            pltpu.VMEM((SCH, SW), jnp.float32),
            pltpu.SemaphoreType.DMA,
            pltpu.SemaphoreType.DMA,
        ],
    )
    return f(s, tgt3)


def _k57_body(x_hbm, tgtf_hbm, s_hbm, tgt3_hbm, mean_hbm, xout_hbm,
              stdout_hbm, acc7, idx_v, mb0, mb1, ub0, ub1, acc5, tb0, tb1,
              xb0, xb1, sem0, sem1, sem2, sem3):
    cid = lax.axis_index("c")
    sid = lax.axis_index("s")
    wid = sid * 2 + cid
    rows_per_tile = N // 16

    def zrow(r, carry):
        for c in range(4):
            mb0[r, pl.ds(c * 16, 16)] = jnp.zeros((16,), jnp.float32)
        return carry

    lax.fori_loop(0, SCH, zrow, 0, unroll=4)
    for z in range(rows_per_tile // SCH):
        pltpu.sync_copy(
            mb0, acc7.at[pl.ds(sid * rows_per_tile + z * SCH, SCH)]
        )
    rem = rows_per_tile % SCH
    pltpu.sync_copy(
        mb0.at[pl.ds(0, rem)],
        acc7.at[pl.ds(sid * rows_per_tile + rows_per_tile - rem, rem)],
    )
    plsc.subcore_barrier()

    def irow(i, carry):
        for c in range(2):
            acc5[c, pl.ds(i * 16, 16)] = jnp.full((16,), FMIN, jnp.float32)
        return carry

    lax.fori_loop(0, N // 16, irow, 0, unroll=8)

    tbufs = (tb0, tb1)
    xbufs = (xb0, xb1)
    tsems = (sem0, sem1)
    xsems = (sem2, sem3)

    for b in range(2):
        pltpu.async_copy(tgtf_hbm.at[pl.ds(b * MCH, MCH)], tbufs[b], tsems[b])
        pltpu.async_copy(
            x_hbm.at[pl.ds(2 * wid, 2), pl.ds(b * MCH, MCH)], xbufs[b],
            xsems[b],
        )

    cvecs = [jnp.full((16,), c, jnp.int32) for c in range(2)]

    def chunk(o, carry):
        for b in range(2):
            q = o * 2 + b
            pltpu.make_async_copy(
                tgtf_hbm.at[pl.ds(q * MCH, MCH)], tbufs[b], tsems[b]
            ).wait()
            pltpu.make_async_copy(
                x_hbm.at[pl.ds(2 * wid, 2), pl.ds(q * MCH, MCH)], xbufs[b],
                xsems[b],
            ).wait()

            def inner(k, carry2):
                pairs = []
                for i in range(BAT):
                    sl = pl.ds((k * BAT + i) * 16, 16)
                    idx = tbufs[b][sl]
                    for c in range(2):
                        pairs.append((cvecs[c], idx, xbufs[b][c, sl]))

                gs = [plsc.load_gather(acc5, [cv, ix]) for cv, ix, _ in pairs]
                for (cv, ix, v), g in zip(pairs, gs):
                    plsc.store_scatter(
                        acc5, [cv, ix], jnp.maximum(g, v), mask=v > g
                    )

                def cond(flag):
                    return flag

                def recheck(flag):
                    g2s = [
                        plsc.load_gather(acc5, [cv, ix])
                        for cv, ix, _ in pairs
                    ]
                    imps = [v > g2 for (_, _, v), g2 in zip(pairs, g2s)]
                    for (cv, ix, v), g2, imp in zip(pairs, g2s, imps):
                        plsc.store_scatter(
                            acc5, [cv, ix], jnp.maximum(g2, v), mask=imp
                        )
                    m = imps[0]
                    for im in imps[1:]:
                        m = jnp.logical_or(m, im)
                    return jnp.any(m)

                lax.while_loop(cond, recheck, True)
                return carry2

            lax.fori_loop(0, (MCH // 16) // BAT, inner, 0)

            @pl.when(o < MNC // 2 - 1)
            def _():
                pltpu.async_copy(
                    tgtf_hbm.at[pl.ds((q + 2) * MCH, MCH)], tbufs[b],
                    tsems[b],
                )
                pltpu.async_copy(
                    x_hbm.at[pl.ds(2 * wid, 2), pl.ds((q + 2) * MCH, MCH)],
                    xbufs[b], xsems[b],
                )
        return carry

    lax.fori_loop(0, MNC // 2, chunk, 0)
    pltpu.sync_copy(acc5, xout_hbm.at[pl.ds(2 * wid, 2)])

    pltpu.sync_copy(tgt3_hbm.at[wid], idx_v)

    mbufs = (mb0, mb1)
    ubufs = (ub0, ub1)
    msems = (sem0, sem1)
    usems = (sem2, sem3)

    for b in range(2):
        pltpu.async_copy(
            s_hbm.at[pl.ds(wid * EW + b * SCH, SCH), pl.ds(64, 64)],
            mbufs[b], msems[b],
        )
        pltpu.async_copy(mean_hbm.at[idx_v.at[b]], ubufs[b], usems[b])

    def sbody(o, carry):
        for b in range(2):
            j = o * 2 + b
            pltpu.make_async_copy(
                s_hbm.at[pl.ds(wid * EW + j * SCH, SCH), pl.ds(64, 64)],
                mbufs[b], msems[b],
            ).wait()
            pltpu.make_async_copy(
                mean_hbm.at[idx_v.at[j]], ubufs[b], usems[b]
            ).wait()

            def crow(r, carry2):
                for c in range(4):
                    sl = pl.ds(c * 16, 16)
                    m = mbufs[b][r, sl]
                    u = ubufs[b][r, sl]
                    mbufs[b][r, sl] = jnp.maximum(m * m - u * u, 0.0) + EPS
                return carry2

            lax.fori_loop(0, SCH, crow, 0, unroll=4)
            pltpu.sync_copy(mbufs[b], acc7.at[idx_v.at[j]], add=True)

            @pl.when(o < SNC // 2 - 1)
            def _():
                pltpu.async_copy(
                    s_hbm.at[pl.ds(wid * EW + (j + 2) * SCH, SCH),
                             pl.ds(64, 64)],
                    mbufs[b], msems[b],
                )
                pltpu.async_copy(
                    mean_hbm.at[idx_v.at[j + 2]], ubufs[b], usems[b]
                )
        return carry

    lax.fori_loop(0, SNC // 2, sbody, 0)
    j_last = SNC - 1
    pltpu.async_copy(
        s_hbm.at[pl.ds(wid * EW + j_last * SCH, SCH), pl.ds(64, 64)],
        mb0, sem0,
    ).wait()
    pltpu.async_copy(mean_hbm.at[idx_v.at[j_last]], ub0, sem2).wait()

    def crow_last(r, carry2):
        for c in range(4):
            sl = pl.ds(c * 16, 16)
            m = mb0[r, sl]
            u = ub0[r, sl]
            mb0[r, sl] = jnp.maximum(m * m - u * u, 0.0) + EPS
        return carry2

    lax.fori_loop(0, SCH, crow_last, 0, unroll=4)
    pltpu.sync_copy(mb0, acc7.at[idx_v.at[j_last]], add=True)

    plsc.subcore_barrier()
    pltpu.sync_copy(
        acc7.at[pl.ds(sid * rows_per_tile, rows_per_tile)],
        stdout_hbm.at[cid, pl.ds(sid * rows_per_tile, rows_per_tile)],
    )


def _k57(x, tgtf, s, tgt3, mean):
    f = pl.kernel(
        _k57_body,
        compiler_params=_SC_PARAMS_NL,
        out_type=(
            jax.ShapeDtypeStruct((64, N), jnp.float32),
            jax.ShapeDtypeStruct((2, N, 64), jnp.float32),
        ),
        mesh=_mesh(),
        scratch_types=[
            pltpu.VMEM_SHARED((N, 64), jnp.float32),
            pltpu.VMEM((SNC, SCH), jnp.int32),
            pltpu.VMEM((SCH, 64), jnp.float32),
            pltpu.VMEM((SCH, 64), jnp.float32),
            pltpu.VMEM((SCH, 64), jnp.float32),
            pltpu.VMEM((SCH, 64), jnp.float32),
            pltpu.VMEM((2, N), jnp.float32),
            pltpu.VMEM((MCH,), jnp.int32),
            pltpu.VMEM((MCH,), jnp.int32),
            pltpu.VMEM((2, MCH), jnp.float32),
            pltpu.VMEM((2, MCH), jnp.float32),
            pltpu.SemaphoreType.DMA,
            pltpu.SemaphoreType.DMA,
            pltpu.SemaphoreType.DMA,
            pltpu.SemaphoreType.DMA,
        ],
    )
    return f(x, tgtf, s, tgt3, mean)


def _k6_body(part_ref, sum_ref, mean_ref):
    s = part_ref[0] + part_ref[1]
    sum_ref[...] = s[:, :64]
    mean_ref[...] = s[:, 64:128] / s[:, 128:129]


def _k6(part):
    nb = 1000
    return pl.pallas_call(
        _k6_body,
        grid=(N // nb,),
        in_specs=[pl.BlockSpec((2, nb, SW), lambda i: (0, i, 0))],
        out_specs=[
            pl.BlockSpec((nb, 64), lambda i: (i, 0)),
            pl.BlockSpec((nb, 64), lambda i: (i, 0)),
        ],
        out_shape=[
            jax.ShapeDtypeStruct((N, 64), jnp.float32),
            jax.ShapeDtypeStruct((N, 64), jnp.float32),
        ],
    )(part)


def _k8_body(sum_ref, mean_ref, sp_ref, x_ref, out_ref):
    std = jnp.sqrt(sp_ref[0] + sp_ref[1])
    out_ref[...] = jnp.concatenate(
        [sum_ref[...], mean_ref[...], std, x_ref[...].T], axis=1
    )


def _k8(sum_agg, mean, stdpart, xacc):
    return pl.pallas_call(
        _k8_body,
        out_shape=jax.ShapeDtypeStruct((N, 4 * 64), jnp.float32),
    )(sum_agg, mean, stdpart, xacc)


def kernel(node_states, adj0, adj1, adj2, adj3, W0, W1):
    adjs = [adj0, adj1, adj2, adj3]
    srcs = [a[:, 0].astype(jnp.int32) for a in adjs]
    tgts = [a[:, 1].astype(jnp.int32) for a in adjs]

    isrc = jnp.concatenate([s + t * N for t, s in enumerate(srcs)])
    itgt = jnp.concatenate([g + (T + t) * N for t, g in enumerate(tgts)])
    idx2 = jnp.stack([isrc, itgt], axis=1).reshape(NW, GNC, GCH)

    tgt_all = jnp.concatenate(tgts)
    tgt3 = tgt_all.reshape(NW, SNC, SCH)

    ab = _k1(node_states, W0)
    tab = ab.reshape(2 * T * N, D)
    p2 = _k2(tab, idx2).reshape(ET, 2 * D)
    s, x = _k3(p2, W1)
    part = _k4(s, tgt3)
    sum_agg, mean = _k6(part)
    xacc, stdpart = _k57(x, tgt_all, s, tgt3, mean)
    return _k8(sum_agg, mean, stdpart, xacc)
